# baseline (speedup 1.0000x reference)
import jax
import jax.numpy as jnp
from jax import lax
from jax.experimental import pallas as pl
from jax.experimental.pallas import tpu as pltpu

_TEMP = 10.0
_NEG = -1e9
_CPAD = 128
_COUNT_LANE = 15


def _loss_body(lab_ref, fs_ref, ft_ref, w_ref, b_ref, out_ref, acc_s, acc_t):
    i = pl.program_id(0)
    nb = pl.num_programs(0)

    @pl.when(i == 0)
    def _init():
        acc_s[...] = jnp.zeros_like(acc_s)
        acc_t[...] = jnp.zeros_like(acc_t)

    w = w_ref[...]
    bias = b_ref[0:1, :]

    fs = fs_ref[...]
    ft = ft_ref[...]
    dn = (((1,), (0,)), ((), ()))
    y_s = lax.dot_general(fs, w, dn, preferred_element_type=jnp.float32) + bias
    y_t = lax.dot_general(ft, w, dn, preferred_element_type=jnp.float32) + bias

    lane = lax.broadcasted_iota(jnp.int32, y_s.shape, 1)

    m_s = jnp.max(y_s, axis=1, keepdims=True)
    e_s = jnp.exp((y_s - m_s) / _TEMP)
    an = e_s / jnp.sqrt(jnp.sum(e_s * e_s, axis=1, keepdims=True))
    an = jnp.where(lane == _COUNT_LANE, 1.0, an)

    lbl = lab_ref[0]
    sub = lax.broadcasted_iota(jnp.int32, (_CPAD, lbl.shape[1]), 0)
    oh_s = (sub == lbl).astype(jnp.float32)
    acc_s[...] += lax.dot_general(
        oh_s, an, (((1,), (0,)), ((), ())), preferred_element_type=jnp.float32)

    m_t = jnp.max(y_t, axis=1, keepdims=True)
    s_t1 = jnp.sum(jnp.exp(y_t - m_t), axis=1, keepdims=True)
    conf = (1.0 / s_t1) > 0.8

    e_t = jnp.exp((y_t - m_t) / _TEMP)
    bn = e_t / jnp.sqrt(jnp.sum(e_t * e_t, axis=1, keepdims=True))
    bn = jnp.where(lane == _COUNT_LANE, 1.0, bn)

    is_max = y_t == m_t
    first = jnp.min(jnp.where(is_max, lane, _CPAD), axis=1, keepdims=True)
    oh_t = ((lane == first) & conf).astype(jnp.float32)
    acc_t[...] += lax.dot_general(
        oh_t, bn, (((0,), (0,)), ((), ())), preferred_element_type=jnp.float32)

    @pl.when(i == nb - 1)
    def _finish():
        prod = acc_s[...] * acc_t[...]
        total = jnp.sum(prod)
        lane2 = lax.broadcasted_iota(jnp.int32, prod.shape, 1)
        count = jnp.sum(jnp.where(lane2 == _COUNT_LANE, prod, 0.0))
        loss_sum = 2.0 * count - total
        out_ref[0, 0] = (_TEMP * _TEMP) * loss_sum / jnp.maximum(count, 1.0)


def kernel(f_s, f_t, label_s, W, b):
    n, d = f_s.shape
    c = W.shape[1]
    rows = 256
    nb = n // rows

    w_pad = jnp.zeros((d, _CPAD), jnp.float32).at[:, :c].set(W)
    b_pad = jnp.full((8, _CPAD), _NEG, jnp.float32).at[0, :c].set(b)
    lab3 = label_s.reshape(nb, 1, rows)

    out = pl.pallas_call(
        _loss_body,
        grid=(nb,),
        in_specs=[
            pl.BlockSpec((1, 1, rows), lambda i: (i, 0, 0)),
            pl.BlockSpec((rows, d), lambda i: (i, 0)),
            pl.BlockSpec((rows, d), lambda i: (i, 0)),
            pl.BlockSpec((d, _CPAD), lambda i: (0, 0)),
            pl.BlockSpec((8, _CPAD), lambda i: (0, 0)),
        ],
        out_specs=pl.BlockSpec(memory_space=pltpu.SMEM),
        out_shape=jax.ShapeDtypeStruct((1, 1), jnp.float32),
        scratch_shapes=[
            pltpu.VMEM((_CPAD, _CPAD), jnp.float32),
            pltpu.VMEM((_CPAD, _CPAD), jnp.float32),
        ],
    )(lab3, f_s, f_t, w_pad, b_pad)
    return out.reshape(1)


# speedup vs baseline: 1.0895x; 1.0895x over previous
import jax
import jax.numpy as jnp
from jax import lax
from jax.experimental import pallas as pl
from jax.experimental.pallas import tpu as pltpu

_TEMP = 10.0


def _loss_body(lab_ref, fs_ref, ft_ref, w_ref, b_ref, out_ref, acc_s, acc_t):
    i = pl.program_id(0)
    nb = pl.num_programs(0)
    c = w_ref.shape[1]
    rows = fs_ref.shape[0]

    @pl.when(i == 0)
    def _init():
        acc_s[...] = jnp.zeros_like(acc_s)
        acc_t[...] = jnp.zeros_like(acc_t)

    w = w_ref[...]
    bias = b_ref[...]

    fs = fs_ref[...]
    ft = ft_ref[...]
    dn = (((1,), (0,)), ((), ()))
    y_s = lax.dot_general(fs, w, dn, preferred_element_type=jnp.float32) + bias
    y_t = lax.dot_general(ft, w, dn, preferred_element_type=jnp.float32) + bias

    ones_col = jnp.ones((rows, 1), jnp.float32)

    m_s = jnp.max(y_s, axis=1, keepdims=True)
    e_s = jnp.exp((y_s - m_s) / _TEMP)
    an = e_s / jnp.sqrt(jnp.sum(e_s * e_s, axis=1, keepdims=True))
    an1 = jnp.concatenate([an, ones_col], axis=1)

    lbl = lab_ref[0]
    sub = lax.broadcasted_iota(jnp.int32, (c + 1, rows), 0)
    oh_s = (sub == lbl).astype(jnp.float32)
    acc_s[...] += lax.dot_general(
        oh_s, an1, (((1,), (0,)), ((), ())), preferred_element_type=jnp.float32)

    m_t = jnp.max(y_t, axis=1, keepdims=True)
    s_t1 = jnp.sum(jnp.exp(y_t - m_t), axis=1, keepdims=True)
    conf = (1.0 / s_t1) > 0.8

    e_t = jnp.exp((y_t - m_t) / _TEMP)
    bn = e_t / jnp.sqrt(jnp.sum(e_t * e_t, axis=1, keepdims=True))
    bn1 = jnp.concatenate([bn, ones_col], axis=1)

    lane = lax.broadcasted_iota(jnp.int32, y_t.shape, 1)
    is_max = y_t == m_t
    first = jnp.min(jnp.where(is_max, lane, c), axis=1, keepdims=True)
    lane1 = lax.broadcasted_iota(jnp.int32, (rows, c + 1), 1)
    oh_t = ((lane1 == first) & conf).astype(jnp.float32)
    acc_t[...] += lax.dot_general(
        oh_t, bn1, (((0,), (0,)), ((), ())), preferred_element_type=jnp.float32)

    @pl.when(i == nb - 1)
    def _finish():
        prod = acc_s[...] * acc_t[...]
        total = jnp.sum(prod)
        lane2 = lax.broadcasted_iota(jnp.int32, prod.shape, 1)
        count = jnp.sum(jnp.where(lane2 == c, prod, 0.0))
        loss_sum = 2.0 * count - total
        out_ref[0, 0] = (_TEMP * _TEMP) * loss_sum / jnp.maximum(count, 1.0)


def kernel(f_s, f_t, label_s, W, b):
    n, d = f_s.shape
    c = W.shape[1]
    rows = 512
    nb = n // rows

    lab3 = label_s.reshape(nb, 1, rows)
    b2 = b.reshape(1, c)

    out = pl.pallas_call(
        _loss_body,
        grid=(nb,),
        in_specs=[
            pl.BlockSpec((1, 1, rows), lambda i: (i, 0, 0)),
            pl.BlockSpec((rows, d), lambda i: (i, 0)),
            pl.BlockSpec((rows, d), lambda i: (i, 0)),
            pl.BlockSpec((d, c), lambda i: (0, 0)),
            pl.BlockSpec((1, c), lambda i: (0, 0)),
        ],
        out_specs=pl.BlockSpec(memory_space=pltpu.SMEM),
        out_shape=jax.ShapeDtypeStruct((1, 1), jnp.float32),
        scratch_shapes=[
            pltpu.VMEM((c + 1, c + 1), jnp.float32),
            pltpu.VMEM((c + 1, c + 1), jnp.float32),
        ],
    )(lab3, f_s, f_t, W, b2)
    return out.reshape(1)


# speedup vs baseline: 1.0897x; 1.0002x over previous
import jax
import jax.numpy as jnp
from jax import lax
from jax.experimental import pallas as pl
from jax.experimental.pallas import tpu as pltpu

_TEMP = 10.0


def _loss_body(lab_ref, fs_ref, ft_ref, w_ref, b_ref, out_ref, acc_s, acc_t):
    i = pl.program_id(0)
    nb = pl.num_programs(0)
    c = w_ref.shape[1]
    rows = fs_ref.shape[0]

    @pl.when(i == 0)
    def _init():
        acc_s[...] = jnp.zeros_like(acc_s)
        acc_t[...] = jnp.zeros_like(acc_t)

    w = w_ref[...]
    bias = b_ref[...]

    fs = fs_ref[...].astype(jnp.bfloat16)
    ft = ft_ref[...].astype(jnp.bfloat16)
    wb = w.astype(jnp.bfloat16)
    dn = (((1,), (0,)), ((), ()))
    y_s = lax.dot_general(fs, wb, dn, preferred_element_type=jnp.float32) + bias
    y_t = lax.dot_general(ft, wb, dn, preferred_element_type=jnp.float32) + bias

    ones_col = jnp.ones((rows, 1), jnp.float32)

    m_s = jnp.max(y_s, axis=1, keepdims=True)
    e_s = jnp.exp((y_s - m_s) / _TEMP)
    an = e_s / jnp.sqrt(jnp.sum(e_s * e_s, axis=1, keepdims=True))
    an1 = jnp.concatenate([an, ones_col], axis=1)

    lbl = lab_ref[0]
    sub = lax.broadcasted_iota(jnp.int32, (c + 1, rows), 0)
    oh_s = (sub == lbl).astype(jnp.float32)
    acc_s[...] += lax.dot_general(
        oh_s, an1, (((1,), (0,)), ((), ())), preferred_element_type=jnp.float32)

    m_t = jnp.max(y_t, axis=1, keepdims=True)
    s_t1 = jnp.sum(jnp.exp(y_t - m_t), axis=1, keepdims=True)
    conf = (1.0 / s_t1) > 0.8

    e_t = jnp.exp((y_t - m_t) / _TEMP)
    bn = e_t / jnp.sqrt(jnp.sum(e_t * e_t, axis=1, keepdims=True))
    bn1 = jnp.concatenate([bn, ones_col], axis=1)

    lane = lax.broadcasted_iota(jnp.int32, y_t.shape, 1)
    is_max = y_t == m_t
    first = jnp.min(jnp.where(is_max, lane, c), axis=1, keepdims=True)
    lane1 = lax.broadcasted_iota(jnp.int32, (rows, c + 1), 1)
    oh_t = ((lane1 == first) & conf).astype(jnp.float32)
    acc_t[...] += lax.dot_general(
        oh_t, bn1, (((0,), (0,)), ((), ())), preferred_element_type=jnp.float32)

    @pl.when(i == nb - 1)
    def _finish():
        prod = acc_s[...] * acc_t[...]
        total = jnp.sum(prod)
        lane2 = lax.broadcasted_iota(jnp.int32, prod.shape, 1)
        count = jnp.sum(jnp.where(lane2 == c, prod, 0.0))
        loss_sum = 2.0 * count - total
        out_ref[0, 0] = (_TEMP * _TEMP) * loss_sum / jnp.maximum(count, 1.0)


def kernel(f_s, f_t, label_s, W, b):
    n, d = f_s.shape
    c = W.shape[1]
    rows = 512
    nb = n // rows

    lab3 = label_s.reshape(nb, 1, rows)
    b2 = b.reshape(1, c)

    out = pl.pallas_call(
        _loss_body,
        grid=(nb,),
        in_specs=[
            pl.BlockSpec((1, 1, rows), lambda i: (i, 0, 0)),
            pl.BlockSpec((rows, d), lambda i: (i, 0)),
            pl.BlockSpec((rows, d), lambda i: (i, 0)),
            pl.BlockSpec((d, c), lambda i: (0, 0)),
            pl.BlockSpec((1, c), lambda i: (0, 0)),
        ],
        out_specs=pl.BlockSpec(memory_space=pltpu.SMEM),
        out_shape=jax.ShapeDtypeStruct((1, 1), jnp.float32),
        scratch_shapes=[
            pltpu.VMEM((c + 1, c + 1), jnp.float32),
            pltpu.VMEM((c + 1, c + 1), jnp.float32),
        ],
    )(lab3, f_s, f_t, W, b2)
    return out.reshape(1)
